# baseline (speedup 1.0000x reference)
import functools

import jax
import jax.numpy as jnp
from jax import lax
from jax.experimental import pallas as pl
from jax.experimental.pallas import tpu as pltpu
from jax.experimental.pallas import tpu_sc as plsc

B = 16384
EMB = 32
CHUNK = 128


def _sc_gather(u2d, i2d, gu_t, gi_t, mu_t, mi_t):
    info = plsc.get_sparse_core_info()
    nw = info.num_cores * info.num_subcores
    nrows = u2d.shape[0]
    cpw = nrows // nw

    mesh = plsc.VectorSubcoreMesh(core_axis_name="c", subcore_axis_name="s")
    out_t = [jax.ShapeDtypeStruct((nrows, CHUNK, EMB), jnp.float32)] * 4
    scratch = [
        pltpu.VMEM((cpw, CHUNK), jnp.int32),
        pltpu.VMEM((cpw, CHUNK), jnp.int32),
        pltpu.VMEM((cpw, CHUNK, EMB), jnp.float32),
        pltpu.VMEM((cpw, CHUNK, EMB), jnp.float32),
        pltpu.VMEM((cpw, CHUNK, EMB), jnp.float32),
        pltpu.VMEM((cpw, CHUNK, EMB), jnp.float32),
        pltpu.SemaphoreType.DMA,
    ]

    @functools.partial(
        pl.kernel, mesh=mesh, out_type=out_t, scratch_types=scratch,
        compiler_params=pltpu.CompilerParams(use_tc_tiling_on_sc=False))
    def body(u_hbm, i_hbm, gu_hbm, gi_hbm, mu_hbm, mi_hbm,
             gu_out, gi_out, mu_out, mi_out,
             uv, iv, bgu, bgi, bmu, bmi, sem):
        wid = lax.axis_index("s") * info.num_cores + lax.axis_index("c")
        row0 = wid * cpw
        pltpu.sync_copy(u_hbm.at[pl.ds(row0, cpw)], uv)
        pltpu.sync_copy(i_hbm.at[pl.ds(row0, cpw)], iv)
        copies = []
        for tbl, idxv, buf in ((gu_hbm, uv, bgu), (gi_hbm, iv, bgi),
                               (mu_hbm, uv, bmu), (mi_hbm, iv, bmi)):
            for j in range(cpw):
                copies.append(
                    pltpu.async_copy(tbl.at[idxv.at[j]], buf.at[j], sem))
        for c in copies:
            c.wait()
        for buf, out in ((bgu, gu_out), (bgi, gi_out),
                         (bmu, mu_out), (bmi, mi_out)):
            pltpu.sync_copy(buf, out.at[pl.ds(row0, cpw)])

    return body(u2d, i2d, gu_t, gi_t, mu_t, mi_t)


def _dense_body(gu_r, gi_r, mu_r, mi_r, w0a_r, w0b_r, b0_r, w1_r, b1_r,
                w2_r, b2_r, wpa_r, wpb_r, bp_r, out_r):
    g = gu_r[...] * gi_r[...]
    h = jnp.dot(mu_r[...], w0a_r[...], preferred_element_type=jnp.float32)
    h = h + jnp.dot(mi_r[...], w0b_r[...], preferred_element_type=jnp.float32)
    h = jnp.maximum(h + b0_r[...], 0.0)
    h = jnp.dot(h, w1_r[...], preferred_element_type=jnp.float32) + b1_r[...]
    h = jnp.maximum(h, 0.0)
    h = jnp.dot(h, w2_r[...], preferred_element_type=jnp.float32) + b2_r[...]
    h = jnp.maximum(h, 0.0)
    o = (jnp.sum(g * wpa_r[...], axis=1) + jnp.sum(h * wpb_r[...], axis=1)
         + bp_r[0, 0])
    out_r[...] = o


def _tc_dense(gu, gi, mu, mi, w0a, w0b, b0, w1, b1, w2, b2, wpa, wpb, bp,
              interpret=False):
    bb = 2048
    grid = (B // bb,)
    data_spec = pl.BlockSpec((bb, EMB), lambda i: (i, 0))

    def full(shape):
        return pl.BlockSpec(shape, lambda i: tuple(0 for _ in shape))

    return pl.pallas_call(
        _dense_body,
        grid=grid,
        in_specs=[
            data_spec, data_spec, data_spec, data_spec,
            full((EMB, 64)), full((EMB, 64)), full((1, 64)),
            full((64, 32)), full((1, 32)),
            full((32, 16)), full((1, 16)),
            full((1, EMB)), full((1, 16)), full((1, 1)),
        ],
        out_specs=pl.BlockSpec((bb,), lambda i: (i,)),
        out_shape=jax.ShapeDtypeStruct((B,), jnp.float32),
        interpret=interpret,
    )(gu, gi, mu, mi, w0a, w0b, b0, w1, b1, w2, b2, wpa, wpb, bp)


def kernel(user_indices, item_indices, gmf_user_table, gmf_item_table,
           mlp_user_table, mlp_item_table, W0, b0, W1, b1, W2, b2, Wp, bp):
    u2d = user_indices.astype(jnp.int32).reshape(-1, CHUNK)
    i2d = item_indices.astype(jnp.int32).reshape(-1, CHUNK)
    gu, gi, mu, mi = _sc_gather(u2d, i2d, gmf_user_table, gmf_item_table,
                                mlp_user_table, mlp_item_table)
    gu = gu.reshape(B, EMB)
    gi = gi.reshape(B, EMB)
    mu = mu.reshape(B, EMB)
    mi = mi.reshape(B, EMB)
    w0a, w0b = W0[:EMB], W0[EMB:]
    wpa = Wp[:EMB, 0].reshape(1, EMB)
    wpb = Wp[EMB:, 0].reshape(1, Wp.shape[0] - EMB)
    return _tc_dense(gu, gi, mu, mi, w0a, w0b, b0.reshape(1, -1),
                     W1, b1.reshape(1, -1), W2, b2.reshape(1, -1),
                     wpa, wpb, bp.reshape(1, 1))
